# Initial kernel scaffold; baseline (speedup 1.0000x reference)
import jax
import jax.numpy as jnp
from jax.experimental import pallas as pl


def kernel(source_points, target_points):
    raise NotImplementedError("write your pallas kernel here")


# speedup vs baseline: 1.6196x; 1.6196x over previous
import jax
import jax.numpy as jnp
from jax.experimental import pallas as pl


_ROWS = 256


def _dist_fine_body(x_ref, yT_ref, rowmin_ref, colmin_ref, amin_ref):
    t = pl.program_id(1)
    x = x_ref[0]
    yT = yT_ref[0]
    m = yT.shape[1]
    x2 = jnp.sum(x * x, axis=1, keepdims=True)
    y2 = jnp.sum(yT * yT, axis=0, keepdims=True)
    ab = jax.lax.dot_general(x, yT, (((1,), (0,)), ((), ())),
                             preferred_element_type=jnp.float32)
    dmat = jnp.maximum(x2 + y2 - 2.0 * ab, 0.0)
    rowmin = jnp.min(dmat, axis=1, keepdims=True)
    rowmin_ref[0] = rowmin
    lane = jax.lax.broadcasted_iota(jnp.int32, dmat.shape, 1)
    amin_ref[0] = jnp.min(jnp.where(dmat == rowmin, lane, m), axis=1,
                          keepdims=True)
    cmin = jnp.min(dmat, axis=0, keepdims=True)

    @pl.when(t == 0)
    def _():
        colmin_ref[0] = cmin

    @pl.when(t != 0)
    def _():
        colmin_ref[0] = jnp.minimum(colmin_ref[0], cmin)


def _dist_coarse_body(x_ref, yT_ref, rowmin_ref, colmin_ref):
    t = pl.program_id(1)
    x = x_ref[0]
    yT = yT_ref[0]
    x2 = jnp.sum(x * x, axis=1, keepdims=True)
    y2 = jnp.sum(yT * yT, axis=0, keepdims=True)
    ab = jax.lax.dot_general(x, yT, (((1,), (0,)), ((), ())),
                             preferred_element_type=jnp.float32)
    dmat = jnp.maximum(x2 + y2 - 2.0 * ab, 0.0)
    rowmin_ref[0] = jnp.min(dmat, axis=1, keepdims=True)
    cmin = jnp.min(dmat, axis=0, keepdims=True)

    @pl.when(t == 0)
    def _():
        colmin_ref[0] = cmin

    @pl.when(t != 0)
    def _():
        colmin_ref[0] = jnp.minimum(colmin_ref[0], cmin)


def _volume(f):
    px, py, pz = f[:, 0, :], f[:, 1, :], f[:, 2, :]
    qx = jnp.roll(px, -1, axis=1)
    qy = jnp.roll(py, -1, axis=1)
    qz = jnp.roll(pz, -1, axis=1)
    rx = jnp.roll(px, -2, axis=1)
    ry = jnp.roll(py, -2, axis=1)
    rz = jnp.roll(pz, -2, axis=1)
    triple = (px * (qy * rz - qz * ry)
              + py * (qz * rx - qx * rz)
              + pz * (qx * ry - qy * rx))
    return jnp.sum(triple, axis=1, keepdims=True) / 6.0


def _loss_body(fT_ref, ntT_ref, rmf_ref, cmf_ref, rmc_ref, cmc_ref, out_ref):
    f = fT_ref[...]
    g = ntT_ref[...]
    la_fine = (jnp.mean(jnp.sqrt(rmf_ref[...]))
               + jnp.mean(jnp.sqrt(cmf_ref[...]))) / 2.0
    la_coarse = (jnp.mean(jnp.sqrt(rmc_ref[...]))
                 + jnp.mean(jnp.sqrt(cmc_ref[...]))) / 2.0
    fz = f[:, 2, :]
    gz = g[:, 2, :]
    in_mag = jnp.sqrt(jnp.sum(fz * fz, axis=1, keepdims=True))
    re_mag = jnp.sqrt(jnp.sum(gz * gz, axis=1, keepdims=True))
    loss_rot = jnp.mean((in_mag - re_mag) ** 2)
    fy = f[:, 1, :]
    gy = g[:, 1, :]
    loss_refl = jnp.mean((fy - gy) ** 2)
    loss_geo = jnp.mean((_volume(f) - _volume(g)) ** 2)
    total = loss_rot + loss_refl + la_coarse + la_fine + loss_geo
    out_ref[...] = total[None, None]


def kernel(source_points, target_points):
    coarse = source_points[0]
    fine = source_points[1]
    B, N, _ = fine.shape
    M = target_points.shape[1]
    R = _ROWS
    T = N // R
    tgtT = jnp.swapaxes(target_points, 1, 2)
    fineT = jnp.swapaxes(fine, 1, 2)

    pts_spec = pl.BlockSpec((1, R, 3), lambda b, t: (b, t, 0))
    tgt_spec = pl.BlockSpec((1, 3, M), lambda b, t: (b, 0, 0))
    rowmin_spec = pl.BlockSpec((1, R, 1), lambda b, t: (b, t, 0))
    colmin_spec = pl.BlockSpec((1, 1, M), lambda b, t: (b, 0, 0))

    rmf, cmf, amin = pl.pallas_call(
        _dist_fine_body,
        grid=(B, T),
        in_specs=[pts_spec, tgt_spec],
        out_specs=[rowmin_spec, colmin_spec, rowmin_spec],
        out_shape=[
            jax.ShapeDtypeStruct((B, N, 1), jnp.float32),
            jax.ShapeDtypeStruct((B, 1, M), jnp.float32),
            jax.ShapeDtypeStruct((B, N, 1), jnp.int32),
        ],
    )(fine, tgtT)

    nt = jnp.take_along_axis(target_points, amin, axis=1)
    ntT = jnp.swapaxes(nt, 1, 2)

    rmc, cmc = pl.pallas_call(
        _dist_coarse_body,
        grid=(B, T),
        in_specs=[pts_spec, tgt_spec],
        out_specs=[rowmin_spec, colmin_spec],
        out_shape=[
            jax.ShapeDtypeStruct((B, N, 1), jnp.float32),
            jax.ShapeDtypeStruct((B, 1, M), jnp.float32),
        ],
    )(coarse, tgtT)

    total = pl.pallas_call(
        _loss_body,
        in_specs=[pl.BlockSpec(a.shape, lambda: (0,) * a.ndim)
                  for a in (fineT, ntT, rmf, cmf, rmc, cmc)],
        out_specs=pl.BlockSpec((1, 1), lambda: (0, 0)),
        out_shape=jax.ShapeDtypeStruct((1, 1), jnp.float32),
    )(fineT, ntT, rmf, cmf, rmc, cmc)

    return total[0, 0]


# speedup vs baseline: 1.7912x; 1.1059x over previous
import jax
import jax.numpy as jnp
from jax.experimental import pallas as pl
from jax.experimental.pallas import tpu as pltpu


_ROWS = 256


def _dist_body(xf_ref, xc_ref, yT_ref,
               amin_ref, chrf_ref, chcf_ref, chrc_ref, chcc_ref,
               cminf_s, cminc_s):
    t = pl.program_id(1)
    nt = pl.num_programs(1)
    yT = yT_ref[0]
    m = yT.shape[1]
    y2 = jnp.sum(yT * yT, axis=0, keepdims=True)

    def one_cloud(x_ref, chr_ref, chc_ref, cmin_s, want_amin):
        x = x_ref[0]
        x2 = jnp.sum(x * x, axis=1, keepdims=True)
        ab = jax.lax.dot_general(x, yT, (((1,), (0,)), ((), ())),
                                 preferred_element_type=jnp.float32)
        d = (x2 + y2) - 2.0 * ab
        rowmin = jnp.min(d, axis=1, keepdims=True)
        if want_amin:
            lane = jax.lax.broadcasted_iota(jnp.int32, d.shape, 1)
            amin_ref[0] = jnp.min(jnp.where(d == rowmin, lane, m), axis=1,
                                  keepdims=True)
        srow = jnp.sum(jnp.sqrt(jnp.maximum(rowmin, 0.0)))
        cmin = jnp.min(d, axis=0, keepdims=True)

        @pl.when(t == 0)
        def _():
            chr_ref[...] = srow[None, None, None]
            cmin_s[...] = cmin

        @pl.when(t != 0)
        def _():
            chr_ref[...] = chr_ref[...] + srow[None, None, None]
            cmin_s[...] = jnp.minimum(cmin_s[...], cmin)

        @pl.when(t == nt - 1)
        def _():
            scol = jnp.sum(jnp.sqrt(jnp.maximum(cmin_s[...], 0.0)))
            chc_ref[...] = scol[None, None, None]

    one_cloud(xf_ref, chrf_ref, chcf_ref, cminf_s, True)
    one_cloud(xc_ref, chrc_ref, chcc_ref, cminc_s, False)


def _volume(f):
    px, py, pz = f[:, 0, :], f[:, 1, :], f[:, 2, :]
    qx = jnp.roll(px, -1, axis=1)
    qy = jnp.roll(py, -1, axis=1)
    qz = jnp.roll(pz, -1, axis=1)
    rx = jnp.roll(px, -2, axis=1)
    ry = jnp.roll(py, -2, axis=1)
    rz = jnp.roll(pz, -2, axis=1)
    triple = (px * (qy * rz - qz * ry)
              + py * (qz * rx - qx * rz)
              + pz * (qx * ry - qy * rx))
    return jnp.sum(triple, axis=1, keepdims=True) / 6.0


def _loss_body(fT_ref, ntT_ref, chrf_ref, chcf_ref, chrc_ref, chcc_ref,
               out_ref):
    f = fT_ref[...]
    g = ntT_ref[...]
    bn = f.shape[0] * f.shape[2]
    la_fine = (jnp.sum(chrf_ref[...]) / bn + jnp.sum(chcf_ref[...]) / bn) / 2.0
    la_coarse = (jnp.sum(chrc_ref[...]) / bn + jnp.sum(chcc_ref[...]) / bn) / 2.0
    fz = f[:, 2, :]
    gz = g[:, 2, :]
    in_mag = jnp.sqrt(jnp.sum(fz * fz, axis=1, keepdims=True))
    re_mag = jnp.sqrt(jnp.sum(gz * gz, axis=1, keepdims=True))
    loss_rot = jnp.mean((in_mag - re_mag) ** 2)
    fy = f[:, 1, :]
    gy = g[:, 1, :]
    loss_refl = jnp.mean((fy - gy) ** 2)
    loss_geo = jnp.mean((_volume(f) - _volume(g)) ** 2)
    total = loss_rot + loss_refl + la_coarse + la_fine + loss_geo
    out_ref[...] = total[None, None]


def kernel(source_points, target_points):
    coarse = source_points[0]
    fine = source_points[1]
    B, N, _ = fine.shape
    M = target_points.shape[1]
    R = _ROWS
    T = N // R
    tgtT = jnp.swapaxes(target_points, 1, 2)
    fineT = jnp.swapaxes(fine, 1, 2)

    pts_spec = pl.BlockSpec((1, R, 3), lambda b, t: (b, t, 0))
    tgt_spec = pl.BlockSpec((1, 3, M), lambda b, t: (b, 0, 0))
    amin_spec = pl.BlockSpec((1, R, 1), lambda b, t: (b, t, 0))
    ch_spec = pl.BlockSpec((1, 1, 1), lambda b, t: (b, 0, 0))
    ch_shape = jax.ShapeDtypeStruct((B, 1, 1), jnp.float32)

    amin, chrf, chcf, chrc, chcc = pl.pallas_call(
        _dist_body,
        grid=(B, T),
        in_specs=[pts_spec, pts_spec, tgt_spec],
        out_specs=[amin_spec, ch_spec, ch_spec, ch_spec, ch_spec],
        out_shape=[
            jax.ShapeDtypeStruct((B, N, 1), jnp.int32),
            ch_shape, ch_shape, ch_shape, ch_shape,
        ],
        scratch_shapes=[
            pltpu.VMEM((1, M), jnp.float32),
            pltpu.VMEM((1, M), jnp.float32),
        ],
        compiler_params=pltpu.CompilerParams(
            dimension_semantics=("arbitrary", "arbitrary")),
    )(fine, coarse, tgtT)

    nt = jnp.take_along_axis(target_points, amin, axis=1)
    ntT = jnp.swapaxes(nt, 1, 2)

    total = pl.pallas_call(
        _loss_body,
        in_specs=[pl.BlockSpec(a.shape, lambda: (0,) * a.ndim)
                  for a in (fineT, ntT, chrf, chcf, chrc, chcc)],
        out_specs=pl.BlockSpec((1, 1), lambda: (0, 0)),
        out_shape=jax.ShapeDtypeStruct((1, 1), jnp.float32),
    )(fineT, ntT, chrf, chcf, chrc, chcc)

    return total[0, 0]


# speedup vs baseline: 2.0246x; 1.1303x over previous
import jax
import jax.numpy as jnp
from jax.experimental import pallas as pl
from jax.experimental.pallas import tpu as pltpu


_ROWS = 512


def _dist_body(xf_ref, xc_ref, yT_ref,
               amin_ref, srf_ref, cmf_ref, src_ref, cmc_ref):
    yT = yT_ref[0]
    m = yT.shape[1]
    y2 = jnp.sum(yT * yT, axis=0, keepdims=True)

    def one_cloud(x_ref, sr_ref, cm_ref, want_amin):
        x = x_ref[0]
        x2 = jnp.sum(x * x, axis=1, keepdims=True)
        ab = jax.lax.dot_general(x, yT, (((1,), (0,)), ((), ())),
                                 preferred_element_type=jnp.float32)
        d = (x2 + y2) - 2.0 * ab
        rowmin = jnp.min(d, axis=1, keepdims=True)
        if want_amin:
            lane = jax.lax.broadcasted_iota(jnp.int32, d.shape, 1)
            amin_ref[0] = jnp.min(jnp.where(d == rowmin, lane, m), axis=1,
                                  keepdims=True)
        srow = jnp.sum(jnp.sqrt(jnp.maximum(rowmin, 0.0)))
        sr_ref[...] = srow[None, None, None, None]
        cm_ref[0, 0] = jnp.min(d, axis=0, keepdims=True)

    one_cloud(xf_ref, srf_ref, cmf_ref, True)
    one_cloud(xc_ref, src_ref, cmc_ref, False)


def _volume(f):
    px, py, pz = f[:, 0, :], f[:, 1, :], f[:, 2, :]
    qx = jnp.roll(px, -1, axis=1)
    qy = jnp.roll(py, -1, axis=1)
    qz = jnp.roll(pz, -1, axis=1)
    rx = jnp.roll(px, -2, axis=1)
    ry = jnp.roll(py, -2, axis=1)
    rz = jnp.roll(pz, -2, axis=1)
    triple = (px * (qy * rz - qz * ry)
              + py * (qz * rx - qx * rz)
              + pz * (qx * ry - qy * rx))
    return jnp.sum(triple, axis=1, keepdims=True) / 6.0


def _loss_body(fT_ref, ntT_ref, srf_ref, cmf_ref, src_ref, cmc_ref,
               out_ref):
    f = fT_ref[...]
    g = ntT_ref[...]
    bn = f.shape[0] * f.shape[2]

    def chamfer(sr_ref, cm_ref):
        colmin = jnp.min(cm_ref[...], axis=1)
        scol = jnp.sum(jnp.sqrt(jnp.maximum(colmin, 0.0)))
        return (jnp.sum(sr_ref[...]) / bn + scol / bn) / 2.0

    la_fine = chamfer(srf_ref, cmf_ref)
    la_coarse = chamfer(src_ref, cmc_ref)
    fz = f[:, 2, :]
    gz = g[:, 2, :]
    in_mag = jnp.sqrt(jnp.sum(fz * fz, axis=1, keepdims=True))
    re_mag = jnp.sqrt(jnp.sum(gz * gz, axis=1, keepdims=True))
    loss_rot = jnp.mean((in_mag - re_mag) ** 2)
    fy = f[:, 1, :]
    gy = g[:, 1, :]
    loss_refl = jnp.mean((fy - gy) ** 2)
    loss_geo = jnp.mean((_volume(f) - _volume(g)) ** 2)
    total = loss_rot + loss_refl + la_coarse + la_fine + loss_geo
    out_ref[...] = total[None, None]


def kernel(source_points, target_points):
    coarse = source_points[0]
    fine = source_points[1]
    B, N, _ = fine.shape
    M = target_points.shape[1]
    R = _ROWS
    T = N // R
    tgtT = jnp.swapaxes(target_points, 1, 2)
    fineT = jnp.swapaxes(fine, 1, 2)

    pts_spec = pl.BlockSpec((1, R, 3), lambda b, t: (b, t, 0))
    tgt_spec = pl.BlockSpec((1, 3, M), lambda b, t: (b, 0, 0))
    amin_spec = pl.BlockSpec((1, R, 1), lambda b, t: (b, t, 0))
    sr_spec = pl.BlockSpec((1, 1, 1, 1), lambda b, t: (b, t, 0, 0))
    cm_spec = pl.BlockSpec((1, 1, 1, M), lambda b, t: (b, t, 0, 0))

    amin, srf, cmf, src, cmc = pl.pallas_call(
        _dist_body,
        grid=(B, T),
        in_specs=[pts_spec, pts_spec, tgt_spec],
        out_specs=[amin_spec, sr_spec, cm_spec, sr_spec, cm_spec],
        out_shape=[
            jax.ShapeDtypeStruct((B, N, 1), jnp.int32),
            jax.ShapeDtypeStruct((B, T, 1, 1), jnp.float32),
            jax.ShapeDtypeStruct((B, T, 1, M), jnp.float32),
            jax.ShapeDtypeStruct((B, T, 1, 1), jnp.float32),
            jax.ShapeDtypeStruct((B, T, 1, M), jnp.float32),
        ],
        compiler_params=pltpu.CompilerParams(
            dimension_semantics=("arbitrary", "arbitrary")),
    )(fine, coarse, tgtT)

    nt = jnp.take_along_axis(target_points, amin, axis=1)
    ntT = jnp.swapaxes(nt, 1, 2)

    total = pl.pallas_call(
        _loss_body,
        in_specs=[pl.BlockSpec(a.shape, lambda n=a.ndim: (0,) * n)
                  for a in (fineT, ntT, srf, cmf, src, cmc)],
        out_specs=pl.BlockSpec((1, 1), lambda: (0, 0)),
        out_shape=jax.ShapeDtypeStruct((1, 1), jnp.float32),
    )(fineT, ntT, srf, cmf, src, cmc)

    return total[0, 0]
